# Initial kernel scaffold; baseline (speedup 1.0000x reference)
import jax
import jax.numpy as jnp
from jax.experimental import pallas as pl


def kernel(h, frac_coords, lattices, frac_diff, W_e1, b_e1, W_e2, b_e2, W_n1, b_n1, W_n2, b_n2, ln_g, ln_b, edge_index, edge2graph):
    raise NotImplementedError("write your pallas kernel here")


# speedup vs baseline: 2.8810x; 2.8810x over previous
import functools

import jax
import jax.numpy as jnp
import numpy as np
from jax import lax
from jax.experimental import pallas as pl
from jax.experimental.pallas import tpu as pltpu
from jax.experimental.pallas import tpu_sc as plsc

N = 10000
E = 320000
G = 64
H = 128
NF = 32

NC = 2
NS = 16
NW = NC * NS
PER_W = E // NW
K = 80
CH = PER_W // K
NPAD = 10240
NPS = NPAD // NS

_f32 = jnp.float32


def _silu(x):
    return x * (1.0 / (1.0 + jnp.exp(-x)))


def _ln_pq_body(h_ref, g_ref, b_ref, w_ref, hn_ref, p_ref, q_ref):
    x = h_ref[...]
    mu = jnp.mean(x, axis=1, keepdims=True)
    xc = x - mu
    var = jnp.mean(xc * xc, axis=1, keepdims=True)
    hn = xc * lax.rsqrt(var + 1e-5) * g_ref[...] + b_ref[...]
    hn_ref[...] = hn
    pq = jnp.dot(hn, w_ref[...], preferred_element_type=_f32)
    p_ref[...] = pq[:, :H]
    q_ref[...] = pq[:, H:]


def _ln_pq(h, ln_g, ln_b, w1ab):
    bn = 1000
    grid = N // bn
    return pl.pallas_call(
        _ln_pq_body,
        grid=(grid,),
        in_specs=[
            pl.BlockSpec((bn, H), lambda i: (i, 0)),
            pl.BlockSpec((1, H), lambda i: (0, 0)),
            pl.BlockSpec((1, H), lambda i: (0, 0)),
            pl.BlockSpec((H, 2 * H), lambda i: (0, 0)),
        ],
        out_specs=[
            pl.BlockSpec((bn, H), lambda i: (i, 0)),
            pl.BlockSpec((bn, H), lambda i: (i, 0)),
            pl.BlockSpec((bn, H), lambda i: (i, 0)),
        ],
        out_shape=[
            jax.ShapeDtypeStruct((N, H), _f32),
            jax.ShapeDtypeStruct((N, H), _f32),
            jax.ShapeDtypeStruct((N, H), _f32),
        ],
    )(h, ln_g, ln_b, w1ab)


@functools.lru_cache(maxsize=None)
def _sc_gather_fn():
    mesh = plsc.VectorSubcoreMesh(core_axis_name="c", subcore_axis_name="s")

    @functools.partial(
        pl.kernel,
        out_type=(
            jax.ShapeDtypeStruct((E, H), _f32),
            jax.ShapeDtypeStruct((E, H), _f32),
        ),
        mesh=mesh,
        scratch_types=[
            pltpu.VMEM((CH, K), jnp.int32),
            pltpu.VMEM((CH, K), jnp.int32),
            pltpu.VMEM((K, H), _f32),
            pltpu.VMEM((K, H), _f32),
            pltpu.SemaphoreType.DMA,
            pltpu.SemaphoreType.DMA,
        ],
    )
    def _sc_gather(src_hbm, dst_hbm, p_hbm, q_hbm, g1_hbm, g2_hbm,
                   si_v, di_v, r1_v, r2_v, sem1, sem2):
        wid = lax.axis_index("s") * NC + lax.axis_index("c")
        base = wid * PER_W
        pltpu.sync_copy(src_hbm.at[wid], si_v)
        pltpu.sync_copy(dst_hbm.at[wid], di_v)

        def body(j, carry):
            row0 = base + j * K
            cp1 = pltpu.async_copy(p_hbm.at[si_v.at[j]], r1_v, sem1)
            cp2 = pltpu.async_copy(q_hbm.at[di_v.at[j]], r2_v, sem2)
            cp1.wait()
            cp2.wait()
            pltpu.sync_copy(r1_v, g1_hbm.at[pl.ds(row0, K)])
            pltpu.sync_copy(r2_v, g2_hbm.at[pl.ds(row0, K)])
            return carry

        lax.fori_loop(0, CH, body, 0)

    return _sc_gather


def _sc_gather_call(src3, dst3, p, q):
    return _sc_gather_fn()(src3, dst3, p, q)


def _edge_body(g1_ref, g2_ref, fd_ref, e2g_ref, lat_ref, w1c_ref, w1d_ref,
               be1_ref, w2_ref, be2_ref, out_ref):
    lat = lat_ref[...]
    cols = []
    for i in range(3):
        for j in range(3):
            acc = lat[:, 3 * i + 0:3 * i + 1] * lat[:, 3 * j + 0:3 * j + 1]
            acc = acc + lat[:, 3 * i + 1:3 * i + 2] * lat[:, 3 * j + 1:3 * j + 2]
            acc = acc + lat[:, 3 * i + 2:3 * i + 3] * lat[:, 3 * j + 2:3 * j + 3]
            cols.append(acc)
    ip9 = jnp.concatenate(cols, axis=1)
    lb = jnp.dot(ip9, w1c_ref[...], preferred_element_type=_f32) + be1_ref[...]

    e2g = e2g_ref[...]
    onehot = (e2g == lax.broadcasted_iota(jnp.int32, (1, G), 1)).astype(_f32)
    lrow = jnp.dot(onehot, lb, preferred_element_type=_f32)

    fd = fd_ref[...]
    freqs = 2.0 * np.pi * lax.broadcasted_iota(jnp.int32, (1, NF), 1).astype(_f32)
    emb = jnp.concatenate([fd[:, d:d + 1] * freqs for d in range(3)], axis=1)
    s192 = jnp.concatenate([jnp.sin(emb), jnp.cos(emb)], axis=1)
    sproj = jnp.dot(s192, w1d_ref[...], preferred_element_type=_f32)

    pre = g1_ref[...] + g2_ref[...] + lrow + sproj
    t1 = _silu(pre)
    out_ref[...] = _silu(
        jnp.dot(t1, w2_ref[...], preferred_element_type=_f32) + be2_ref[...])


def _edge_mlp(g1, g2, fd, e2g, lat9, w1c, w1d, be1, w2, be2):
    be = 2000
    grid = E // be
    return pl.pallas_call(
        _edge_body,
        grid=(grid,),
        in_specs=[
            pl.BlockSpec((be, H), lambda i: (i, 0)),
            pl.BlockSpec((be, H), lambda i: (i, 0)),
            pl.BlockSpec((be, 3), lambda i: (i, 0)),
            pl.BlockSpec((be, 1), lambda i: (i, 0)),
            pl.BlockSpec((G, 9), lambda i: (0, 0)),
            pl.BlockSpec((9, H), lambda i: (0, 0)),
            pl.BlockSpec((2 * 3 * NF, H), lambda i: (0, 0)),
            pl.BlockSpec((1, H), lambda i: (0, 0)),
            pl.BlockSpec((H, H), lambda i: (0, 0)),
            pl.BlockSpec((1, H), lambda i: (0, 0)),
        ],
        out_specs=pl.BlockSpec((be, H), lambda i: (i, 0)),
        out_shape=jax.ShapeDtypeStruct((E, H), _f32),
    )(g1, g2, fd, e2g, lat9, w1c, w1d, be1, w2, be2)


NH = NPAD // H


@functools.lru_cache(maxsize=None)
def _sc_scatter_fn():
    mesh = plsc.VectorSubcoreMesh(core_axis_name="c", subcore_axis_name="s")

    @functools.partial(
        pl.kernel,
        out_type=(
            jax.ShapeDtypeStruct((NC, NPAD, H), _f32),
            jax.ShapeDtypeStruct((NC, NH, H), _f32),
        ),
        mesh=mesh,
        scratch_types=[
            pltpu.VMEM((CH, K), jnp.int32),
            pltpu.VMEM((K, H), _f32),
            pltpu.VMEM((NH, H), _f32),
            pltpu.VMEM((NH,), jnp.int32),
            pltpu.VMEM_SHARED((NPAD, H), _f32),
            pltpu.VMEM_SHARED((NH, H), _f32),
        ],
        compiler_params=pltpu.CompilerParams(needs_layout_passes=False),
    )
    def _sc_scatter(src_hbm, e_hbm, zs_hbm, zh_hbm, i80_hbm, sums_hbm, cnt_hbm,
                    idx_v, rows_v, hist_v, i80_v, acc_sh, cnt_sh):
        c = lax.axis_index("c")
        s = lax.axis_index("s")
        wid = s * NC + c
        base = wid * PER_W

        pltpu.sync_copy(zs_hbm.at[pl.ds(s * NPS, NPS)],
                        acc_sh.at[pl.ds(s * NPS, NPS)])
        pltpu.sync_copy(zh_hbm, hist_v)
        pltpu.sync_copy(i80_hbm, i80_v)

        @pl.when(s == 0)
        def _():
            pltpu.sync_copy(zh_hbm, cnt_sh)

        pltpu.sync_copy(src_hbm.at[wid], idx_v)
        plsc.subcore_barrier()
        ones16 = jnp.full((16,), 1.0, _f32)

        def body(j, carry):
            pltpu.sync_copy(e_hbm.at[pl.ds(base + j * K, K)], rows_v)
            pltpu.sync_copy(rows_v, acc_sh.at[idx_v.at[j]], add=True)

            def hist(l, carry2):
                iv = idx_v[j, pl.ds(l * 16, 16)]
                plsc.addupdate_scatter(hist_v, [iv >> 7, iv & 127], ones16)
                return carry2

            lax.fori_loop(0, K // 16, hist, 0)
            return carry

        lax.fori_loop(0, CH, body, 0)
        pltpu.sync_copy(hist_v, cnt_sh.at[i80_v], add=True)
        plsc.subcore_barrier()

        pltpu.sync_copy(acc_sh.at[pl.ds(s * NPS, NPS)],
                        sums_hbm.at[c, pl.ds(s * NPS, NPS)])

        @pl.when(s == 0)
        def _():
            pltpu.sync_copy(cnt_sh, cnt_hbm.at[c])

    return _sc_scatter


def _sc_scatter_call(src3, e, zs, zh, i80):
    return _sc_scatter_fn()(src3, e, zs, zh, i80)


def _node_body(h_ref, hn_ref, s_ref, c_ref, w1_ref, b1_ref, w2_ref, b2_ref,
               o_ref):
    sm = s_ref[0] + s_ref[1]
    cn = c_ref[0] + c_ref[1]
    agg = sm / jnp.maximum(cn, 1.0)
    nin = jnp.concatenate([hn_ref[...], agg], axis=1)
    t = _silu(jnp.dot(nin, w1_ref[...], preferred_element_type=_f32) + b1_ref[...])
    o = _silu(jnp.dot(t, w2_ref[...], preferred_element_type=_f32) + b2_ref[...])
    o_ref[...] = h_ref[...] + o


def _node_mlp(h, hn, sums_p, cnt_p, w1, b1, w2, b2):
    bn = 1000
    grid = N // bn
    return pl.pallas_call(
        _node_body,
        grid=(grid,),
        in_specs=[
            pl.BlockSpec((bn, H), lambda i: (i, 0)),
            pl.BlockSpec((bn, H), lambda i: (i, 0)),
            pl.BlockSpec((NC, bn, H), lambda i: (0, i, 0)),
            pl.BlockSpec((NC, bn, 1), lambda i: (0, i, 0)),
            pl.BlockSpec((2 * H, H), lambda i: (0, 0)),
            pl.BlockSpec((1, H), lambda i: (0, 0)),
            pl.BlockSpec((H, H), lambda i: (0, 0)),
            pl.BlockSpec((1, H), lambda i: (0, 0)),
        ],
        out_specs=pl.BlockSpec((bn, H), lambda i: (i, 0)),
        out_shape=jax.ShapeDtypeStruct((N, H), _f32),
    )(h, hn, sums_p, cnt_p, w1, b1, w2, b2)


def kernel(h, frac_coords, lattices, frac_diff,
           W_e1, b_e1, W_e2, b_e2, W_n1, b_n1, W_n2, b_n2,
           ln_g, ln_b, edge_index, edge2graph):
    w1ab = jnp.concatenate([W_e1[:H], W_e1[H:2 * H]], axis=1)
    w1c = W_e1[2 * H:2 * H + 9]
    w1d = W_e1[2 * H + 9:]

    hn, p, q = _ln_pq(h, ln_g.reshape(1, H), ln_b.reshape(1, H), w1ab)

    src = edge_index[0].astype(jnp.int32)
    dst = edge_index[1].astype(jnp.int32)
    src3 = src.reshape(NW, CH, K)
    dst3 = dst.reshape(NW, CH, K)

    g1, g2 = _sc_gather_call(src3, dst3, p, q)

    e = _edge_mlp(g1, g2, frac_diff, edge2graph.astype(jnp.int32).reshape(E, 1),
                  lattices.reshape(G, 9), w1c, w1d, b_e1.reshape(1, H),
                  W_e2, b_e2.reshape(1, H))

    zs = jnp.zeros((NPAD, H), _f32)
    zh = jnp.zeros((NH, H), _f32)
    i80 = jnp.arange(NH, dtype=jnp.int32)
    sums_p, cnt_p = _sc_scatter_call(src3, e, zs, zh, i80)
    cnt = cnt_p.reshape(NC, NPAD)[:, :N].reshape(NC, N, 1)

    out = _node_mlp(h, hn, sums_p, cnt,
                    W_n1, b_n1.reshape(1, H), W_n2, b_n2.reshape(1, H))
    return out


# speedup vs baseline: 4.7022x; 1.6321x over previous
import functools

import jax
import jax.numpy as jnp
import numpy as np
from jax import lax
from jax.experimental import pallas as pl
from jax.experimental.pallas import tpu as pltpu
from jax.experimental.pallas import tpu_sc as plsc

N = 10000
E = 320000
G = 64
H = 128
NF = 32

NC = 2
NS = 16
NW = NC * NS
PER_W = E // NW
K = 80
CH = PER_W // K
NPAD = 10240
NPS = NPAD // NS

_f32 = jnp.float32


def _silu(x):
    return x * (1.0 / (1.0 + jnp.exp(-x)))


def _ln_pq_body(h_ref, g_ref, b_ref, w_ref, hn_ref, p_ref, q_ref):
    x = h_ref[...]
    mu = jnp.mean(x, axis=1, keepdims=True)
    xc = x - mu
    var = jnp.mean(xc * xc, axis=1, keepdims=True)
    hn = xc * lax.rsqrt(var + 1e-5) * g_ref[...] + b_ref[...]
    hn_ref[...] = hn
    pq = jnp.dot(hn, w_ref[...], preferred_element_type=_f32)
    p_ref[...] = pq[:, :H]
    q_ref[...] = pq[:, H:]


def _ln_pq(h, ln_g, ln_b, w1ab):
    bn = 1000
    grid = N // bn
    return pl.pallas_call(
        _ln_pq_body,
        grid=(grid,),
        in_specs=[
            pl.BlockSpec((bn, H), lambda i: (i, 0)),
            pl.BlockSpec((1, H), lambda i: (0, 0)),
            pl.BlockSpec((1, H), lambda i: (0, 0)),
            pl.BlockSpec((H, 2 * H), lambda i: (0, 0)),
        ],
        out_specs=[
            pl.BlockSpec((bn, H), lambda i: (i, 0)),
            pl.BlockSpec((bn, H), lambda i: (i, 0)),
            pl.BlockSpec((bn, H), lambda i: (i, 0)),
        ],
        out_shape=[
            jax.ShapeDtypeStruct((N, H), _f32),
            jax.ShapeDtypeStruct((N, H), _f32),
            jax.ShapeDtypeStruct((N, H), _f32),
        ],
    )(h, ln_g, ln_b, w1ab)


@functools.lru_cache(maxsize=None)
def _sc_gather_fn():
    mesh = plsc.VectorSubcoreMesh(core_axis_name="c", subcore_axis_name="s")

    @functools.partial(
        pl.kernel,
        out_type=(
            jax.ShapeDtypeStruct((E, H), _f32),
            jax.ShapeDtypeStruct((E, H), _f32),
        ),
        mesh=mesh,
        scratch_types=[
            pltpu.VMEM((CH, K), jnp.int32),
            pltpu.VMEM((CH, K), jnp.int32),
            pltpu.VMEM((2, K, H), _f32),
            pltpu.VMEM((2, K, H), _f32),
            pltpu.SemaphoreType.DMA((2,)),
            pltpu.SemaphoreType.DMA((2,)),
            pltpu.SemaphoreType.DMA((2,)),
            pltpu.SemaphoreType.DMA((2,)),
        ],
    )
    def _sc_gather(src_hbm, dst_hbm, p_hbm, q_hbm, g1_hbm, g2_hbm,
                   si_v, di_v, r1_v, r2_v, sg1, sg2, sw1, sw2):
        wid = lax.axis_index("s") * NC + lax.axis_index("c")
        base = wid * PER_W
        pltpu.sync_copy(src_hbm.at[wid], si_v)
        pltpu.sync_copy(dst_hbm.at[wid], di_v)

        def g_start(j, b):
            pltpu.make_async_copy(p_hbm.at[si_v.at[j]], r1_v.at[b], sg1.at[b]).start()
            pltpu.make_async_copy(q_hbm.at[di_v.at[j]], r2_v.at[b], sg2.at[b]).start()

        def g_wait(j, b):
            pltpu.make_async_copy(p_hbm.at[si_v.at[j]], r1_v.at[b], sg1.at[b]).wait()
            pltpu.make_async_copy(q_hbm.at[di_v.at[j]], r2_v.at[b], sg2.at[b]).wait()

        def w_start(j, b):
            row0 = base + j * K
            pltpu.make_async_copy(r1_v.at[b], g1_hbm.at[pl.ds(row0, K)], sw1.at[b]).start()
            pltpu.make_async_copy(r2_v.at[b], g2_hbm.at[pl.ds(row0, K)], sw2.at[b]).start()

        def w_wait(j, b):
            row0 = base + j * K
            pltpu.make_async_copy(r1_v.at[b], g1_hbm.at[pl.ds(row0, K)], sw1.at[b]).wait()
            pltpu.make_async_copy(r2_v.at[b], g2_hbm.at[pl.ds(row0, K)], sw2.at[b]).wait()

        g_start(0, 0)
        g_start(1, 1)

        def body(j2, carry):
            j = 2 * j2
            g_wait(j, 0)
            w_start(j, 0)
            g_wait(j + 1, 1)
            w_start(j + 1, 1)
            w_wait(j, 0)

            @pl.when(j2 < CH // 2 - 1)
            def _():
                g_start(j + 2, 0)

            w_wait(j + 1, 1)

            @pl.when(j2 < CH // 2 - 1)
            def _():
                g_start(j + 3, 1)

            return carry

        lax.fori_loop(0, CH // 2, body, 0)
        jl = CH - 1
        g_start(jl, 0)
        g_wait(jl, 0)
        pltpu.sync_copy(r1_v.at[0], g1_hbm.at[pl.ds(base + jl * K, K)])
        pltpu.sync_copy(r2_v.at[0], g2_hbm.at[pl.ds(base + jl * K, K)])

    return _sc_gather


def _sc_gather_call(src3, dst3, p, q):
    return _sc_gather_fn()(src3, dst3, p, q)


_SIN_C = (6.2831836, -41.34148, 81.59766, -76.594925, 41.269928, -12.372495)
_COS_C = (1.0, -19.739206, 64.93917, -85.451164, 60.17623, -26.000528,
          6.5756116)


def _edge_body(g1_ref, g2_ref, fd_ref, e2g_ref, lat_ref, kmat_ref, w1c_ref,
               w1d_ref, be1_ref, w2_ref, be2_ref, out_ref):
    lat = lat_ref[...]
    cols = []
    for i in range(3):
        for j in range(3):
            acc = lat[:, 3 * i + 0:3 * i + 1] * lat[:, 3 * j + 0:3 * j + 1]
            acc = acc + lat[:, 3 * i + 1:3 * i + 2] * lat[:, 3 * j + 1:3 * j + 2]
            acc = acc + lat[:, 3 * i + 2:3 * i + 3] * lat[:, 3 * j + 2:3 * j + 3]
            cols.append(acc)
    ip9 = jnp.concatenate(cols, axis=1)
    lb = jnp.dot(ip9, w1c_ref[...], preferred_element_type=_f32) + be1_ref[...]

    e2g = e2g_ref[...]
    onehot = (e2g == lax.broadcasted_iota(jnp.int32, (1, G), 1)).astype(_f32)
    lrow = jnp.dot(onehot, lb, preferred_element_type=_f32)

    fd = fd_ref[...]
    f = jnp.dot(fd, kmat_ref[...], preferred_element_type=_f32)
    u = f - jnp.round(f)
    u2 = u * u
    sp = _SIN_C[5]
    for a in (_SIN_C[4], _SIN_C[3], _SIN_C[2], _SIN_C[1], _SIN_C[0]):
        sp = sp * u2 + a
    sv = u * sp
    cp = _COS_C[6]
    for a in (_COS_C[5], _COS_C[4], _COS_C[3], _COS_C[2], _COS_C[1], _COS_C[0]):
        cp = cp * u2 + a
    s192 = jnp.concatenate([sv, cp], axis=1)
    sproj = jnp.dot(s192, w1d_ref[...], preferred_element_type=_f32)

    pre = g1_ref[...] + g2_ref[...] + lrow + sproj
    t1 = _silu(pre)
    out_ref[...] = _silu(
        jnp.dot(t1, w2_ref[...], preferred_element_type=_f32) + be2_ref[...])


def _edge_mlp(g1, g2, fd, e2g, lat9, kmat, w1c, w1d, be1, w2, be2):
    be = 2000
    grid = E // be
    return pl.pallas_call(
        _edge_body,
        grid=(grid,),
        in_specs=[
            pl.BlockSpec((be, H), lambda i: (i, 0)),
            pl.BlockSpec((be, H), lambda i: (i, 0)),
            pl.BlockSpec((be, 3), lambda i: (i, 0)),
            pl.BlockSpec((be, 1), lambda i: (i, 0)),
            pl.BlockSpec((G, 9), lambda i: (0, 0)),
            pl.BlockSpec((3, 3 * NF), lambda i: (0, 0)),
            pl.BlockSpec((9, H), lambda i: (0, 0)),
            pl.BlockSpec((2 * 3 * NF, H), lambda i: (0, 0)),
            pl.BlockSpec((1, H), lambda i: (0, 0)),
            pl.BlockSpec((H, H), lambda i: (0, 0)),
            pl.BlockSpec((1, H), lambda i: (0, 0)),
        ],
        out_specs=pl.BlockSpec((be, H), lambda i: (i, 0)),
        out_shape=jax.ShapeDtypeStruct((E, H), _f32),
    )(g1, g2, fd, e2g, lat9, kmat, w1c, w1d, be1, w2, be2)


NH = NPAD // H


@functools.lru_cache(maxsize=None)
def _sc_scatter_fn():
    mesh = plsc.VectorSubcoreMesh(core_axis_name="c", subcore_axis_name="s")

    @functools.partial(
        pl.kernel,
        out_type=(
            jax.ShapeDtypeStruct((NC, NPAD, H), _f32),
            jax.ShapeDtypeStruct((NC, NH, H), _f32),
        ),
        mesh=mesh,
        scratch_types=[
            pltpu.VMEM((CH, K), jnp.int32),
            pltpu.VMEM((2, K, H), _f32),
            pltpu.VMEM((NH, H), _f32),
            pltpu.VMEM((NH,), jnp.int32),
            pltpu.VMEM_SHARED((NPAD, H), _f32),
            pltpu.VMEM_SHARED((NH, H), _f32),
            pltpu.SemaphoreType.DMA((2,)),
        ],
        compiler_params=pltpu.CompilerParams(needs_layout_passes=False),
    )
    def _sc_scatter(src_hbm, e_hbm, zs_hbm, zh_hbm, i80_hbm, sums_hbm, cnt_hbm,
                    idx_v, rows_v, hist_v, i80_v, acc_sh, cnt_sh, sl):
        c = lax.axis_index("c")
        s = lax.axis_index("s")
        wid = s * NC + c
        base = wid * PER_W

        pltpu.sync_copy(zs_hbm.at[pl.ds(s * NPS, NPS)],
                        acc_sh.at[pl.ds(s * NPS, NPS)])
        pltpu.sync_copy(zh_hbm, hist_v)
        pltpu.sync_copy(i80_hbm, i80_v)

        @pl.when(s == 0)
        def _():
            pltpu.sync_copy(zh_hbm, cnt_sh)

        pltpu.sync_copy(src_hbm.at[wid], idx_v)
        plsc.subcore_barrier()
        ones16 = jnp.full((16,), 1.0, _f32)

        def l_start(j, b):
            pltpu.make_async_copy(e_hbm.at[pl.ds(base + j * K, K)],
                                  rows_v.at[b], sl.at[b]).start()

        def l_wait(j, b):
            pltpu.make_async_copy(e_hbm.at[pl.ds(base + j * K, K)],
                                  rows_v.at[b], sl.at[b]).wait()

        def hist(j):
            def h1(l, carry2):
                iv = idx_v[j, pl.ds(l * 16, 16)]
                plsc.addupdate_scatter(hist_v, [iv >> 7, iv & 127], ones16)
                return carry2

            lax.fori_loop(0, K // 16, h1, 0)

        l_start(0, 0)

        def body(j2, carry):
            j = 2 * j2
            l_wait(j, 0)

            @pl.when(j2 < CH // 2)
            def _():
                l_start(j + 1, 1)

            pltpu.sync_copy(rows_v.at[0], acc_sh.at[idx_v.at[j]], add=True)
            hist(j)

            @pl.when(j2 < CH // 2)
            def _():
                l_wait(j + 1, 1)
                l_start(j + 2, 0)
                pltpu.sync_copy(rows_v.at[1], acc_sh.at[idx_v.at[j + 1]], add=True)
                hist(j + 1)

            return carry

        lax.fori_loop(0, (CH + 1) // 2, body, 0)
        pltpu.sync_copy(hist_v, cnt_sh.at[i80_v], add=True)
        plsc.subcore_barrier()

        pltpu.sync_copy(acc_sh.at[pl.ds(s * NPS, NPS)],
                        sums_hbm.at[c, pl.ds(s * NPS, NPS)])

        @pl.when(s == 0)
        def _():
            pltpu.sync_copy(cnt_sh, cnt_hbm.at[c])

    return _sc_scatter


def _sc_scatter_call(src3, e, zs, zh, i80):
    return _sc_scatter_fn()(src3, e, zs, zh, i80)


def _node_body(h_ref, hn_ref, s_ref, c_ref, w1_ref, b1_ref, w2_ref, b2_ref,
               o_ref):
    sm = s_ref[0] + s_ref[1]
    cn = c_ref[0] + c_ref[1]
    agg = sm / jnp.maximum(cn, 1.0)
    nin = jnp.concatenate([hn_ref[...], agg], axis=1)
    t = _silu(jnp.dot(nin, w1_ref[...], preferred_element_type=_f32) + b1_ref[...])
    o = _silu(jnp.dot(t, w2_ref[...], preferred_element_type=_f32) + b2_ref[...])
    o_ref[...] = h_ref[...] + o


def _node_mlp(h, hn, sums_p, cnt_p, w1, b1, w2, b2):
    bn = 1000
    grid = N // bn
    return pl.pallas_call(
        _node_body,
        grid=(grid,),
        in_specs=[
            pl.BlockSpec((bn, H), lambda i: (i, 0)),
            pl.BlockSpec((bn, H), lambda i: (i, 0)),
            pl.BlockSpec((NC, bn, H), lambda i: (0, i, 0)),
            pl.BlockSpec((NC, bn, 1), lambda i: (0, i, 0)),
            pl.BlockSpec((2 * H, H), lambda i: (0, 0)),
            pl.BlockSpec((1, H), lambda i: (0, 0)),
            pl.BlockSpec((H, H), lambda i: (0, 0)),
            pl.BlockSpec((1, H), lambda i: (0, 0)),
        ],
        out_specs=pl.BlockSpec((bn, H), lambda i: (i, 0)),
        out_shape=jax.ShapeDtypeStruct((N, H), _f32),
    )(h, hn, sums_p, cnt_p, w1, b1, w2, b2)


def kernel(h, frac_coords, lattices, frac_diff,
           W_e1, b_e1, W_e2, b_e2, W_n1, b_n1, W_n2, b_n2,
           ln_g, ln_b, edge_index, edge2graph):
    w1ab = jnp.concatenate([W_e1[:H], W_e1[H:2 * H]], axis=1)
    w1c = W_e1[2 * H:2 * H + 9]
    w1d = W_e1[2 * H + 9:]

    hn, p, q = _ln_pq(h, ln_g.reshape(1, H), ln_b.reshape(1, H), w1ab)

    src = edge_index[0].astype(jnp.int32)
    dst = edge_index[1].astype(jnp.int32)
    src3 = src.reshape(NW, CH, K)
    dst3 = dst.reshape(NW, CH, K)

    g1, g2 = _sc_gather_call(src3, dst3, p, q)

    kmat = np.zeros((3, 3 * NF), np.float32)
    for d in range(3):
        kmat[d, d * NF:(d + 1) * NF] = np.arange(NF, dtype=np.float32)
    e = _edge_mlp(g1, g2, frac_diff, edge2graph.astype(jnp.int32).reshape(E, 1),
                  lattices.reshape(G, 9), jnp.asarray(kmat), w1c, w1d,
                  b_e1.reshape(1, H), W_e2, b_e2.reshape(1, H))

    zs = jnp.zeros((NPAD, H), _f32)
    zh = jnp.zeros((NH, H), _f32)
    i80 = jnp.arange(NH, dtype=jnp.int32)
    sums_p, cnt_p = _sc_scatter_call(src3, e, zs, zh, i80)
    cnt = cnt_p.reshape(NC, NPAD)[:, :N].reshape(NC, N, 1)

    out = _node_mlp(h, hn, sums_p, cnt,
                    W_n1, b_n1.reshape(1, H), W_n2, b_n2.reshape(1, H))
    return out


# speedup vs baseline: 5.0652x; 1.0772x over previous
import functools

import jax
import jax.numpy as jnp
import numpy as np
from jax import lax
from jax.experimental import pallas as pl
from jax.experimental.pallas import tpu as pltpu
from jax.experimental.pallas import tpu_sc as plsc

N = 10000
E = 320000
G = 64
H = 128
NF = 32

NC = 2
NS = 16
NW = NC * NS
PER_W = E // NW
K = 80
CH = PER_W // K
NPAD = 10240
NPS = NPAD // NS

_f32 = jnp.float32


def _silu(x):
    return x * (1.0 / (1.0 + jnp.exp(-x)))


def _ln_pq_body(h_ref, g_ref, b_ref, w_ref, hn_ref, p_ref, q_ref):
    x = h_ref[...]
    mu = jnp.mean(x, axis=1, keepdims=True)
    xc = x - mu
    var = jnp.mean(xc * xc, axis=1, keepdims=True)
    hn = xc * lax.rsqrt(var + 1e-5) * g_ref[...] + b_ref[...]
    hn_ref[...] = hn
    pq = jnp.dot(hn, w_ref[...], preferred_element_type=_f32)
    p_ref[...] = pq[:, :H]
    q_ref[...] = pq[:, H:]


def _ln_pq(h, ln_g, ln_b, w1ab):
    bn = 1000
    grid = N // bn
    return pl.pallas_call(
        _ln_pq_body,
        grid=(grid,),
        in_specs=[
            pl.BlockSpec((bn, H), lambda i: (i, 0)),
            pl.BlockSpec((1, H), lambda i: (0, 0)),
            pl.BlockSpec((1, H), lambda i: (0, 0)),
            pl.BlockSpec((H, 2 * H), lambda i: (0, 0)),
        ],
        out_specs=[
            pl.BlockSpec((bn, H), lambda i: (i, 0)),
            pl.BlockSpec((bn, H), lambda i: (i, 0)),
            pl.BlockSpec((bn, H), lambda i: (i, 0)),
        ],
        out_shape=[
            jax.ShapeDtypeStruct((N, H), _f32),
            jax.ShapeDtypeStruct((N, H), _f32),
            jax.ShapeDtypeStruct((N, H), _f32),
        ],
    )(h, ln_g, ln_b, w1ab)


@functools.lru_cache(maxsize=None)
def _sc_gather_fn():
    mesh = plsc.VectorSubcoreMesh(core_axis_name="c", subcore_axis_name="s")

    @functools.partial(
        pl.kernel,
        out_type=jax.ShapeDtypeStruct((E, H // 2), jnp.int32),
        mesh=mesh,
        scratch_types=[
            pltpu.VMEM((CH, K), jnp.int32),
            pltpu.VMEM((CH, K), jnp.int32),
            pltpu.VMEM((2, K, H), _f32),
            pltpu.VMEM((2, K, H), _f32),
            pltpu.VMEM((2, K, H // 2), jnp.int32),
            pltpu.SemaphoreType.DMA((2,)),
            pltpu.SemaphoreType.DMA((2,)),
            pltpu.SemaphoreType.DMA((2,)),
        ],
        compiler_params=pltpu.CompilerParams(needs_layout_passes=False),
    )
    def _sc_gather(src_hbm, dst_hbm, p_hbm, q_hbm, g12_hbm,
                   si_v, di_v, r1_v, r2_v, o_v, sg1, sg2, sw):
        wid = lax.axis_index("s") * NC + lax.axis_index("c")
        base = wid * PER_W
        pltpu.sync_copy(src_hbm.at[wid], si_v)
        pltpu.sync_copy(dst_hbm.at[wid], di_v)

        def g_start(j, b):
            pltpu.make_async_copy(p_hbm.at[si_v.at[j]], r1_v.at[b], sg1.at[b]).start()
            pltpu.make_async_copy(q_hbm.at[di_v.at[j]], r2_v.at[b], sg2.at[b]).start()

        def g_wait(j, b):
            pltpu.make_async_copy(p_hbm.at[si_v.at[j]], r1_v.at[b], sg1.at[b]).wait()
            pltpu.make_async_copy(q_hbm.at[di_v.at[j]], r2_v.at[b], sg2.at[b]).wait()

        def w_start(j, b):
            row0 = base + j * K
            pltpu.make_async_copy(o_v.at[b], g12_hbm.at[pl.ds(row0, K)], sw.at[b]).start()

        def w_wait(j, b):
            row0 = base + j * K
            pltpu.make_async_copy(o_v.at[b], g12_hbm.at[pl.ds(row0, K)], sw.at[b]).wait()

        def sum_pack(b):
            def row(r, carry):
                def grp(g, carry2):
                    a = r1_v[b, r, pl.ds(g * 16, 16)] + r2_v[b, r, pl.ds(g * 16, 16)]
                    bb = (r1_v[b, r, pl.ds(64 + g * 16, 16)]
                          + r2_v[b, r, pl.ds(64 + g * 16, 16)])
                    w = plsc.bitcast(plsc.pack(a, bb, format=plsc.PackFormat.INTERLEAVED), jnp.int32)
                    o_v[b, r, pl.ds(g * 16, 16)] = w
                    return carry2

                lax.fori_loop(0, 4, grp, 0)
                return carry

            lax.fori_loop(0, K, row, 0)

        g_start(0, 0)
        g_start(1, 1)

        def body(j2, carry):
            j = 2 * j2
            g_wait(j, 0)

            @pl.when(j2 > 0)
            def _():
                w_wait(j - 2, 0)

            sum_pack(0)
            w_start(j, 0)

            @pl.when(j2 < CH // 2 - 1)
            def _():
                g_start(j + 2, 0)

            g_wait(j + 1, 1)

            @pl.when(j2 > 0)
            def _():
                w_wait(j - 1, 1)

            sum_pack(1)
            w_start(j + 1, 1)

            @pl.when(j2 < CH // 2 - 1)
            def _():
                g_start(j + 3, 1)

            return carry

        lax.fori_loop(0, CH // 2, body, 0)
        jl = CH - 1
        w_wait(jl - 2, 0)
        g_start(jl, 0)
        g_wait(jl, 0)
        sum_pack(0)
        pltpu.sync_copy(o_v.at[0], g12_hbm.at[pl.ds(base + jl * K, K)])
        w_wait(jl - 1, 1)

    return _sc_gather


def _sc_gather_call(src3, dst3, p, q):
    return _sc_gather_fn()(src3, dst3, p, q)


_SIN_C = (6.2831836, -41.34148, 81.59766, -76.594925, 41.269928, -12.372495)
_COS_C = (1.0, -19.739206, 64.93917, -85.451164, 60.17623, -26.000528,
          6.5756116)


def _edge_body(g12_ref, fd_ref, e2g_ref, lat_ref, kmat_ref, w1c_ref,
               w1d_ref, be1_ref, w2_ref, be2_ref, out_ref):
    lat = lat_ref[...]
    cols = []
    for i in range(3):
        for j in range(3):
            acc = lat[:, 3 * i + 0:3 * i + 1] * lat[:, 3 * j + 0:3 * j + 1]
            acc = acc + lat[:, 3 * i + 1:3 * i + 2] * lat[:, 3 * j + 1:3 * j + 2]
            acc = acc + lat[:, 3 * i + 2:3 * i + 3] * lat[:, 3 * j + 2:3 * j + 3]
            cols.append(acc)
    ip9 = jnp.concatenate(cols, axis=1)
    lb = jnp.dot(ip9, w1c_ref[...], preferred_element_type=_f32) + be1_ref[...]

    e2g = e2g_ref[...]
    onehot = (e2g == lax.broadcasted_iota(jnp.int32, (1, G), 1)).astype(_f32)
    lrow = jnp.dot(onehot, lb, preferred_element_type=_f32)

    fd = fd_ref[...]
    f = jnp.dot(fd, kmat_ref[...], preferred_element_type=_f32)
    u = f - jnp.round(f)
    u2 = u * u
    sp = _SIN_C[5]
    for a in (_SIN_C[4], _SIN_C[3], _SIN_C[2], _SIN_C[1], _SIN_C[0]):
        sp = sp * u2 + a
    sv = u * sp
    cp = _COS_C[6]
    for a in (_COS_C[5], _COS_C[4], _COS_C[3], _COS_C[2], _COS_C[1], _COS_C[0]):
        cp = cp * u2 + a
    s192 = jnp.concatenate([sv, cp], axis=1)
    sproj = jnp.dot(s192, w1d_ref[...], preferred_element_type=_f32)

    x = g12_ref[...]
    lo = lax.bitcast_convert_type(x << 16, _f32)
    hi = lax.bitcast_convert_type(x & jnp.int32(-65536), _f32)
    pre = jnp.concatenate([lo, hi], axis=1) + lrow + sproj
    t1 = _silu(pre)
    out_ref[...] = _silu(
        jnp.dot(t1, w2_ref[...], preferred_element_type=_f32) + be2_ref[...])


def _edge_mlp(g12, fd, e2g, lat9, kmat, w1c, w1d, be1, w2, be2):
    be = 2000
    grid = E // be
    return pl.pallas_call(
        _edge_body,
        grid=(grid,),
        in_specs=[
            pl.BlockSpec((be, H // 2), lambda i: (i, 0)),
            pl.BlockSpec((be, 3), lambda i: (i, 0)),
            pl.BlockSpec((be, 1), lambda i: (i, 0)),
            pl.BlockSpec((G, 9), lambda i: (0, 0)),
            pl.BlockSpec((3, 3 * NF), lambda i: (0, 0)),
            pl.BlockSpec((9, H), lambda i: (0, 0)),
            pl.BlockSpec((2 * 3 * NF, H), lambda i: (0, 0)),
            pl.BlockSpec((1, H), lambda i: (0, 0)),
            pl.BlockSpec((H, H), lambda i: (0, 0)),
            pl.BlockSpec((1, H), lambda i: (0, 0)),
        ],
        out_specs=pl.BlockSpec((be, H), lambda i: (i, 0)),
        out_shape=jax.ShapeDtypeStruct((E, H), _f32),
    )(g12, fd, e2g, lat9, kmat, w1c, w1d, be1, w2, be2)


NH = NPAD // H


@functools.lru_cache(maxsize=None)
def _sc_scatter_fn():
    mesh = plsc.VectorSubcoreMesh(core_axis_name="c", subcore_axis_name="s")

    @functools.partial(
        pl.kernel,
        out_type=(
            jax.ShapeDtypeStruct((NC, NPAD, H), _f32),
            jax.ShapeDtypeStruct((NC, NH, H), _f32),
        ),
        mesh=mesh,
        scratch_types=[
            pltpu.VMEM((CH, K), jnp.int32),
            pltpu.VMEM((2, K, H), _f32),
            pltpu.VMEM((NH, H), _f32),
            pltpu.VMEM((NH,), jnp.int32),
            pltpu.VMEM_SHARED((NPAD, H), _f32),
            pltpu.VMEM_SHARED((NH, H), _f32),
            pltpu.SemaphoreType.DMA((2,)),
        ],
        compiler_params=pltpu.CompilerParams(needs_layout_passes=False),
    )
    def _sc_scatter(src_hbm, e_hbm, zs_hbm, zh_hbm, i80_hbm, sums_hbm, cnt_hbm,
                    idx_v, rows_v, hist_v, i80_v, acc_sh, cnt_sh, sl):
        c = lax.axis_index("c")
        s = lax.axis_index("s")
        wid = s * NC + c
        base = wid * PER_W

        pltpu.sync_copy(zs_hbm.at[pl.ds(s * NPS, NPS)],
                        acc_sh.at[pl.ds(s * NPS, NPS)])
        pltpu.sync_copy(zh_hbm, hist_v)
        pltpu.sync_copy(i80_hbm, i80_v)

        @pl.when(s == 0)
        def _():
            pltpu.sync_copy(zh_hbm, cnt_sh)

        pltpu.sync_copy(src_hbm.at[wid], idx_v)
        plsc.subcore_barrier()
        ones16 = jnp.full((16,), 1.0, _f32)

        def l_start(j, b):
            pltpu.make_async_copy(e_hbm.at[pl.ds(base + j * K, K)],
                                  rows_v.at[b], sl.at[b]).start()

        def l_wait(j, b):
            pltpu.make_async_copy(e_hbm.at[pl.ds(base + j * K, K)],
                                  rows_v.at[b], sl.at[b]).wait()

        def hist(j):
            def h1(l, carry2):
                iv = idx_v[j, pl.ds(l * 16, 16)]
                plsc.addupdate_scatter(hist_v, [iv >> 7, iv & 127], ones16)
                return carry2

            lax.fori_loop(0, K // 16, h1, 0)

        l_start(0, 0)

        def body(j2, carry):
            j = 2 * j2
            l_wait(j, 0)

            @pl.when(j2 < CH // 2)
            def _():
                l_start(j + 1, 1)

            pltpu.sync_copy(rows_v.at[0], acc_sh.at[idx_v.at[j]], add=True)
            hist(j)

            @pl.when(j2 < CH // 2)
            def _():
                l_wait(j + 1, 1)
                l_start(j + 2, 0)
                pltpu.sync_copy(rows_v.at[1], acc_sh.at[idx_v.at[j + 1]], add=True)
                hist(j + 1)

            return carry

        lax.fori_loop(0, (CH + 1) // 2, body, 0)
        pltpu.sync_copy(hist_v, cnt_sh.at[i80_v], add=True)
        plsc.subcore_barrier()

        pltpu.sync_copy(acc_sh.at[pl.ds(s * NPS, NPS)],
                        sums_hbm.at[c, pl.ds(s * NPS, NPS)])

        @pl.when(s == 0)
        def _():
            pltpu.sync_copy(cnt_sh, cnt_hbm.at[c])

    return _sc_scatter


def _sc_scatter_call(src3, e, zs, zh, i80):
    return _sc_scatter_fn()(src3, e, zs, zh, i80)


def _node_body(h_ref, hn_ref, s_ref, c_ref, w1_ref, b1_ref, w2_ref, b2_ref,
               o_ref):
    sm = s_ref[0] + s_ref[1]
    cn = c_ref[0] + c_ref[1]
    agg = sm / jnp.maximum(cn, 1.0)
    nin = jnp.concatenate([hn_ref[...], agg], axis=1)
    t = _silu(jnp.dot(nin, w1_ref[...], preferred_element_type=_f32) + b1_ref[...])
    o = _silu(jnp.dot(t, w2_ref[...], preferred_element_type=_f32) + b2_ref[...])
    o_ref[...] = h_ref[...] + o


def _node_mlp(h, hn, sums_p, cnt_p, w1, b1, w2, b2):
    bn = 1000
    grid = N // bn
    return pl.pallas_call(
        _node_body,
        grid=(grid,),
        in_specs=[
            pl.BlockSpec((bn, H), lambda i: (i, 0)),
            pl.BlockSpec((bn, H), lambda i: (i, 0)),
            pl.BlockSpec((NC, bn, H), lambda i: (0, i, 0)),
            pl.BlockSpec((NC, bn, 1), lambda i: (0, i, 0)),
            pl.BlockSpec((2 * H, H), lambda i: (0, 0)),
            pl.BlockSpec((1, H), lambda i: (0, 0)),
            pl.BlockSpec((H, H), lambda i: (0, 0)),
            pl.BlockSpec((1, H), lambda i: (0, 0)),
        ],
        out_specs=pl.BlockSpec((bn, H), lambda i: (i, 0)),
        out_shape=jax.ShapeDtypeStruct((N, H), _f32),
    )(h, hn, sums_p, cnt_p, w1, b1, w2, b2)


def kernel(h, frac_coords, lattices, frac_diff,
           W_e1, b_e1, W_e2, b_e2, W_n1, b_n1, W_n2, b_n2,
           ln_g, ln_b, edge_index, edge2graph):
    w1ab = jnp.concatenate([W_e1[:H], W_e1[H:2 * H]], axis=1)
    w1c = W_e1[2 * H:2 * H + 9]
    w1d = W_e1[2 * H + 9:]

    hn, p, q = _ln_pq(h, ln_g.reshape(1, H), ln_b.reshape(1, H), w1ab)

    src = edge_index[0].astype(jnp.int32)
    dst = edge_index[1].astype(jnp.int32)
    src3 = src.reshape(NW, CH, K)
    dst3 = dst.reshape(NW, CH, K)

    g12 = _sc_gather_call(src3, dst3, p, q)

    kmat = np.zeros((3, 3 * NF), np.float32)
    for d in range(3):
        kmat[d, d * NF:(d + 1) * NF] = np.arange(NF, dtype=np.float32)
    e = _edge_mlp(g12, frac_diff, edge2graph.astype(jnp.int32).reshape(E, 1),
                  lattices.reshape(G, 9), jnp.asarray(kmat), w1c, w1d,
                  b_e1.reshape(1, H), W_e2, b_e2.reshape(1, H))

    zs = jnp.zeros((NPAD, H), _f32)
    zh = jnp.zeros((NH, H), _f32)
    i80 = jnp.arange(NH, dtype=jnp.int32)
    sums_p, cnt_p = _sc_scatter_call(src3, e, zs, zh, i80)
    cnt = cnt_p.reshape(NC, NPAD)[:, :N].reshape(NC, N, 1)

    out = _node_mlp(h, hn, sums_p, cnt,
                    W_n1, b_n1.reshape(1, H), W_n2, b_n2.reshape(1, H))
    return out
